# baseline (speedup 1.0000x reference)
import functools
import math

import jax
import jax.numpy as jnp
from jax import lax
from jax.experimental import pallas as pl
from jax.experimental.pallas import tpu as pltpu
from jax.experimental.pallas import tpu_sc as plsc


def _bf16_bits(x_f32):
    b = jax.lax.bitcast_convert_type(x_f32.astype(jnp.bfloat16), jnp.uint16)
    return b.astype(jnp.int32)


def _kproj_body(hs_ref, hd_ref, wkse_ref, wkso_ref, wkde_ref, wkdo_ref,
                out_ref):
    hs = hs_ref[...].astype(jnp.bfloat16)
    hd = hd_ref[...].astype(jnp.bfloat16)
    ke = (jnp.dot(hs, wkse_ref[...].astype(jnp.bfloat16),
                  preferred_element_type=jnp.float32)
          + jnp.dot(hd, wkde_ref[...].astype(jnp.bfloat16),
                    preferred_element_type=jnp.float32))
    ko = (jnp.dot(hs, wkso_ref[...].astype(jnp.bfloat16),
                  preferred_element_type=jnp.float32)
          + jnp.dot(hd, wkdo_ref[...].astype(jnp.bfloat16),
                    preferred_element_type=jnp.float32))
    lo = _bf16_bits(ke)
    hi = _bf16_bits(ko)
    out_ref[...] = jax.lax.bitwise_or(jax.lax.shift_left(hi, 16), lo)


def _kproj(h_s, h_d, wks_e, wks_o, wkd_e, wkd_o, block_n=512):
    n, h = h_s.shape
    h2 = h // 2
    grid = (n // block_n,)
    return pl.pallas_call(
        _kproj_body,
        grid=grid,
        in_specs=[
            pl.BlockSpec((block_n, h), lambda i: (i, 0)),
            pl.BlockSpec((block_n, h), lambda i: (i, 0)),
            pl.BlockSpec((h, h2), lambda i: (0, 0)),
            pl.BlockSpec((h, h2), lambda i: (0, 0)),
            pl.BlockSpec((h, h2), lambda i: (0, 0)),
            pl.BlockSpec((h, h2), lambda i: (0, 0)),
        ],
        out_specs=pl.BlockSpec((block_n, h2), lambda i: (i, 0)),
        out_shape=jax.ShapeDtypeStruct((n, h2), jnp.int32),
    )(h_s, h_d, wks_e, wks_o, wkd_e, wkd_o)


def _sc_gather(table, idx_flat, chunk=128):
    n_rows = idx_flat.shape[0]
    h = table.shape[1]
    dt = table.dtype
    info = plsc.get_sparse_core_info()
    nw = info.num_cores * info.num_subcores
    rows_per_w = n_rows // nw
    n_chunks = rows_per_w // chunk
    mesh = plsc.VectorSubcoreMesh(core_axis_name="c", subcore_axis_name="s")

    nbuf = 3

    @functools.partial(
        pl.kernel,
        mesh=mesh,
        out_type=jax.ShapeDtypeStruct((n_rows, h), dt),
        scratch_types=[
            pltpu.VMEM((rows_per_w,), jnp.int32),
            pltpu.VMEM((chunk, h), dt),
            pltpu.VMEM((chunk, h), dt),
            pltpu.VMEM((chunk, h), dt),
            pltpu.SemaphoreType.DMA,
            pltpu.SemaphoreType.DMA,
            pltpu.SemaphoreType.DMA,
            pltpu.SemaphoreType.DMA,
        ],
    )
    def gather_kernel(table_hbm, idx_hbm, out_hbm, idx_v, rows_a, rows_b,
                      rows_c, gsem, osem_a, osem_b, osem_c):
        wid = lax.axis_index("s") * info.num_cores + lax.axis_index("c")
        base = wid * rows_per_w
        pltpu.sync_copy(idx_hbm.at[pl.ds(base, rows_per_w)], idx_v)
        bufs = (rows_a, rows_b, rows_c)
        osems = (osem_a, osem_b, osem_c)
        out_handles = [None] * nbuf
        for i in range(n_chunks):
            s = i % nbuf
            if out_handles[s] is not None:
                out_handles[s].wait()
            off = base + i * chunk
            pltpu.async_copy(
                table_hbm.at[idx_v.at[pl.ds(i * chunk, chunk)]], bufs[s], gsem
            ).wait()
            out_handles[s] = pltpu.async_copy(
                bufs[s], out_hbm.at[pl.ds(off, chunk)], osems[s])
        for hnd in out_handles:
            if hnd is not None:
                hnd.wait()

    return gather_kernel(table, idx_flat)


def _attn_body(ki_ref, hq_ref, wqe_ref, wqo_ref, g_ref, nidx_ref, nlen_ref,
               cur_ref, act_ref, lp_ref, *, m, h):
    bb = hq_ref.shape[0]
    hq = hq_ref[...].astype(jnp.bfloat16)
    qe = jnp.dot(hq, wqe_ref[...].astype(jnp.bfloat16),
                 preferred_element_type=jnp.float32)
    qo = jnp.dot(hq, wqo_ref[...].astype(jnp.bfloat16),
                 preferred_element_type=jnp.float32)
    w = ki_ref[...]
    lo = jax.lax.bitcast_convert_type(
        jax.lax.bitwise_and(w, 0xFFFF).astype(jnp.uint16), jnp.bfloat16
    ).astype(jnp.float32)
    hi = jax.lax.bitcast_convert_type(
        jax.lax.shift_right_logical(w, 16).astype(jnp.uint16), jnp.bfloat16
    ).astype(jnp.float32)
    qe_r = qe.astype(jnp.bfloat16).astype(jnp.float32)
    qo_r = qo.astype(jnp.bfloat16).astype(jnp.float32)
    compat = (
        jnp.sum(lo * qe_r[:, None, :], axis=-1)
        + jnp.sum(hi * qo_r[:, None, :], axis=-1)
    ) / math.sqrt(h)
    nlen = nlen_ref[...]
    lane = lax.broadcasted_iota(jnp.int32, (bb, m), 1)
    mask = lane < nlen
    logits = jnp.where(mask, compat, -1e9)
    z = logits + g_ref[...]
    idx = jnp.argmax(z, axis=1)
    mx = jnp.max(logits, axis=1, keepdims=True)
    shifted = logits - mx
    logp_all = shifted - jnp.log(jnp.sum(jnp.exp(shifted), axis=1, keepdims=True))
    sel = lane == idx[:, None]
    logp = jnp.sum(jnp.where(sel, logp_all, 0.0), axis=1)
    chosen = jnp.sum(jnp.where(sel, nidx_ref[...], 0), axis=1)
    empty = nlen[:, 0] == 0
    act_ref[...] = jnp.where(empty, cur_ref[...][:, 0], chosen)[:, None]
    lp_ref[...] = jnp.where(empty, 0.0, logp)[:, None]


def _attn_sample(ki, hq, wq_e, wq_o, gumbel, neigh_idx, neigh_len,
                 current_nodes, block_b=256):
    b, m = neigh_idx.shape
    h = hq.shape[1]
    h2 = h // 2
    grid = (b // block_b,)
    return pl.pallas_call(
        functools.partial(_attn_body, m=m, h=h),
        grid=grid,
        in_specs=[
            pl.BlockSpec((block_b, m, h2), lambda i: (i, 0, 0)),
            pl.BlockSpec((block_b, h), lambda i: (i, 0)),
            pl.BlockSpec((h, h2), lambda i: (0, 0)),
            pl.BlockSpec((h, h2), lambda i: (0, 0)),
            pl.BlockSpec((block_b, m), lambda i: (i, 0)),
            pl.BlockSpec((block_b, m), lambda i: (i, 0)),
            pl.BlockSpec((block_b, 1), lambda i: (i, 0)),
            pl.BlockSpec((block_b, 1), lambda i: (i, 0)),
        ],
        out_specs=[
            pl.BlockSpec((block_b, 1), lambda i: (i, 0)),
            pl.BlockSpec((block_b, 1), lambda i: (i, 0)),
        ],
        out_shape=[
            jax.ShapeDtypeStruct((b, 1), jnp.int32),
            jax.ShapeDtypeStruct((b, 1), jnp.float32),
        ],
    )(ki, hq, wq_e, wq_o, gumbel, neigh_idx, neigh_len, current_nodes)


def kernel(h_dynamic, h_static, W_skvl, W_dkvl, W_q, current_nodes, neigh_idx,
           neigh_len):
    n, h = h_static.shape
    b, m = neigh_idx.shape
    h2 = h // 2
    wks_e = W_skvl[:, 0:h2]
    wks_o = W_skvl[:, h2:h]
    wkd_e = W_dkvl[:, 0:h2]
    wkd_o = W_dkvl[:, h2:h]
    wq_e = W_q[:, 0:h2]
    wq_o = W_q[:, h2:h]

    kall_i32 = _kproj(h_static, h_dynamic, wks_e, wks_o, wkd_e, wkd_o)

    cur = current_nodes.astype(jnp.int32)
    hq = h_static[cur] + h_dynamic[cur]

    u = jax.random.uniform(jax.random.key(42), (b, m), minval=1e-9, maxval=1.0)
    gumbel = -jnp.log(-jnp.log(u))

    idx_flat = neigh_idx.reshape(b * m).astype(jnp.int32)
    nidx32 = neigh_idx.astype(jnp.int32)
    nlen2 = neigh_len.reshape(b, 1).astype(jnp.int32)
    cur2 = cur.reshape(b, 1)

    nsplit = 2
    bh = b // nsplit
    acts, lps = [], []
    kis = [
        _sc_gather(kall_i32, idx_flat[i * bh * m:(i + 1) * bh * m])
        .reshape(bh, m, h // 2)
        for i in range(nsplit)
    ]
    for i in range(nsplit):
        sl = slice(i * bh, (i + 1) * bh)
        a2, l2 = _attn_sample(
            kis[i], hq[sl], wq_e, wq_o, gumbel[sl],
            nidx32[sl], nlen2[sl], cur2[sl],
        )
        acts.append(a2[:, 0])
        lps.append(l2[:, 0])
    return jnp.concatenate(acts), jnp.concatenate(lps)


# speedup vs baseline: 1.0293x; 1.0293x over previous
import functools
import math

import jax
import jax.numpy as jnp
import numpy as np
from jax import lax
from jax.experimental import pallas as pl
from jax.experimental.pallas import tpu as pltpu
from jax.experimental.pallas import tpu_sc as plsc

_GUMBEL_CACHE = {}


def _gumbel_const(b, m):
    if (b, m) not in _GUMBEL_CACHE:
        with jax.ensure_compile_time_eval():
            u = jax.random.uniform(jax.random.key(42), (b, m), minval=1e-9,
                                   maxval=1.0)
            g = -jnp.log(-jnp.log(u))
        _GUMBEL_CACHE[(b, m)] = np.asarray(g)
    return _GUMBEL_CACHE[(b, m)]


def _bf16_bits(x_f32):
    b = jax.lax.bitcast_convert_type(x_f32.astype(jnp.bfloat16), jnp.uint16)
    return b.astype(jnp.int32)


def _kproj_body(hs_ref, hd_ref, wkse_ref, wkso_ref, wkde_ref, wkdo_ref,
                out_ref):
    hs = hs_ref[...].astype(jnp.bfloat16)
    hd = hd_ref[...].astype(jnp.bfloat16)
    ke = (jnp.dot(hs, wkse_ref[...].astype(jnp.bfloat16),
                  preferred_element_type=jnp.float32)
          + jnp.dot(hd, wkde_ref[...].astype(jnp.bfloat16),
                    preferred_element_type=jnp.float32))
    ko = (jnp.dot(hs, wkso_ref[...].astype(jnp.bfloat16),
                  preferred_element_type=jnp.float32)
          + jnp.dot(hd, wkdo_ref[...].astype(jnp.bfloat16),
                    preferred_element_type=jnp.float32))
    lo = _bf16_bits(ke)
    hi = _bf16_bits(ko)
    out_ref[...] = jax.lax.bitwise_or(jax.lax.shift_left(hi, 16), lo)


def _kproj(h_s, h_d, w_skvl, w_dkvl, block_n=512):
    n, h = h_s.shape
    h2 = h // 2
    grid = (n // block_n,)
    return pl.pallas_call(
        _kproj_body,
        grid=grid,
        in_specs=[
            pl.BlockSpec((block_n, h), lambda i: (i, 0)),
            pl.BlockSpec((block_n, h), lambda i: (i, 0)),
            pl.BlockSpec((h, h2), lambda i: (0, 0)),
            pl.BlockSpec((h, h2), lambda i: (0, 1)),
            pl.BlockSpec((h, h2), lambda i: (0, 0)),
            pl.BlockSpec((h, h2), lambda i: (0, 1)),
        ],
        out_specs=pl.BlockSpec((block_n, h2), lambda i: (i, 0)),
        out_shape=jax.ShapeDtypeStruct((n, h2), jnp.int32),
    )(h_s, h_d, w_skvl, w_skvl, w_dkvl, w_dkvl)


def _sc_gather(table, idx_flat, chunk=128):
    n_rows = idx_flat.shape[0]
    h = table.shape[1]
    dt = table.dtype
    info = plsc.get_sparse_core_info()
    nw = info.num_cores * info.num_subcores
    rows_per_w = n_rows // nw
    n_chunks = rows_per_w // chunk
    mesh = plsc.VectorSubcoreMesh(core_axis_name="c", subcore_axis_name="s")

    nbuf = 3

    @functools.partial(
        pl.kernel,
        mesh=mesh,
        out_type=jax.ShapeDtypeStruct((n_rows, h), dt),
        scratch_types=[
            pltpu.VMEM((rows_per_w,), jnp.int32),
            pltpu.VMEM((chunk, h), dt),
            pltpu.VMEM((chunk, h), dt),
            pltpu.VMEM((chunk, h), dt),
            pltpu.SemaphoreType.DMA,
            pltpu.SemaphoreType.DMA,
            pltpu.SemaphoreType.DMA,
            pltpu.SemaphoreType.DMA,
        ],
    )
    def gather_kernel(table_hbm, idx_hbm, out_hbm, idx_v, rows_a, rows_b,
                      rows_c, gsem, osem_a, osem_b, osem_c):
        wid = lax.axis_index("s") * info.num_cores + lax.axis_index("c")
        base = wid * rows_per_w
        pltpu.sync_copy(idx_hbm.at[pl.ds(base, rows_per_w)], idx_v)
        bufs = (rows_a, rows_b, rows_c)
        osems = (osem_a, osem_b, osem_c)
        out_handles = [None] * nbuf
        for i in range(n_chunks):
            s = i % nbuf
            if out_handles[s] is not None:
                out_handles[s].wait()
            off = base + i * chunk
            pltpu.async_copy(
                table_hbm.at[idx_v.at[pl.ds(i * chunk, chunk)]], bufs[s], gsem
            ).wait()
            out_handles[s] = pltpu.async_copy(
                bufs[s], out_hbm.at[pl.ds(off, chunk)], osems[s])
        for hnd in out_handles:
            if hnd is not None:
                hnd.wait()

    return gather_kernel(table, idx_flat)


def _attn_body(ki_ref, hs_ref, hd_ref, wqe_ref, wqo_ref, g_ref, nidx_ref,
               nlen_ref, cur_ref, act_ref, lp_ref, *, m, h):
    bb = hs_ref.shape[0]
    hq = (hs_ref[...] + hd_ref[...]).astype(jnp.bfloat16)
    qe = jnp.dot(hq, wqe_ref[...].astype(jnp.bfloat16),
                 preferred_element_type=jnp.float32)
    qo = jnp.dot(hq, wqo_ref[...].astype(jnp.bfloat16),
                 preferred_element_type=jnp.float32)
    w = ki_ref[...]
    lo = jax.lax.bitcast_convert_type(
        jax.lax.bitwise_and(w, 0xFFFF).astype(jnp.uint16), jnp.bfloat16
    ).astype(jnp.float32)
    hi = jax.lax.bitcast_convert_type(
        jax.lax.shift_right_logical(w, 16).astype(jnp.uint16), jnp.bfloat16
    ).astype(jnp.float32)
    qe_r = qe.astype(jnp.bfloat16).astype(jnp.float32)
    qo_r = qo.astype(jnp.bfloat16).astype(jnp.float32)
    compat = (
        jnp.sum(lo * qe_r[:, None, :], axis=-1)
        + jnp.sum(hi * qo_r[:, None, :], axis=-1)
    ) / math.sqrt(h)
    nlen = nlen_ref[...]
    lane = lax.broadcasted_iota(jnp.int32, (bb, m), 1)
    mask = lane < nlen
    logits = jnp.where(mask, compat, -1e9)
    z = logits + g_ref[...]
    idx = jnp.argmax(z, axis=1)
    mx = jnp.max(logits, axis=1, keepdims=True)
    shifted = logits - mx
    logp_all = shifted - jnp.log(jnp.sum(jnp.exp(shifted), axis=1, keepdims=True))
    sel = lane == idx[:, None]
    logp = jnp.sum(jnp.where(sel, logp_all, 0.0), axis=1)
    chosen = jnp.sum(jnp.where(sel, nidx_ref[...], 0), axis=1)
    empty = nlen[:, 0] == 0
    act_ref[...] = jnp.where(empty, cur_ref[...][:, 0], chosen)[:, None]
    lp_ref[...] = jnp.where(empty, 0.0, logp)[:, None]


def _attn_sample(ki, hs_cur, hd_cur, wq, gumbel, neigh_idx, neigh_len,
                 current_nodes, block_b=256):
    b, m = neigh_idx.shape
    h = hs_cur.shape[1]
    h2 = h // 2
    grid = (b // block_b,)
    return pl.pallas_call(
        functools.partial(_attn_body, m=m, h=h),
        grid=grid,
        in_specs=[
            pl.BlockSpec((block_b, m, h2), lambda i: (i, 0, 0)),
            pl.BlockSpec((block_b, h), lambda i: (i, 0)),
            pl.BlockSpec((block_b, h), lambda i: (i, 0)),
            pl.BlockSpec((h, h2), lambda i: (0, 0)),
            pl.BlockSpec((h, h2), lambda i: (0, 1)),
            pl.BlockSpec((block_b, m), lambda i: (i, 0)),
            pl.BlockSpec((block_b, m), lambda i: (i, 0)),
            pl.BlockSpec((block_b, 1), lambda i: (i, 0)),
            pl.BlockSpec((block_b, 1), lambda i: (i, 0)),
        ],
        out_specs=[
            pl.BlockSpec((block_b, 1), lambda i: (i, 0)),
            pl.BlockSpec((block_b, 1), lambda i: (i, 0)),
        ],
        out_shape=[
            jax.ShapeDtypeStruct((b, 1), jnp.int32),
            jax.ShapeDtypeStruct((b, 1), jnp.float32),
        ],
    )(ki, hs_cur, hd_cur, wq, wq, gumbel, neigh_idx, neigh_len,
      current_nodes)


def kernel(h_dynamic, h_static, W_skvl, W_dkvl, W_q, current_nodes, neigh_idx,
           neigh_len):
    n, h = h_static.shape
    b, m = neigh_idx.shape
    kall_i32 = _kproj(h_static, h_dynamic, W_skvl, W_dkvl)

    cur = current_nodes.astype(jnp.int32)
    hs_cur = h_static[cur]
    hd_cur = h_dynamic[cur]

    gumbel = jnp.asarray(_gumbel_const(b, m))

    idx_flat = neigh_idx.reshape(b * m).astype(jnp.int32)
    nidx32 = neigh_idx.astype(jnp.int32)
    nlen2 = neigh_len.reshape(b, 1).astype(jnp.int32)
    cur2 = cur.reshape(b, 1)

    nsplit = 2
    bh = b // nsplit
    acts, lps = [], []
    kis = [
        _sc_gather(kall_i32, idx_flat[i * bh * m:(i + 1) * bh * m])
        .reshape(bh, m, h // 2)
        for i in range(nsplit)
    ]
    for i in range(nsplit):
        sl = slice(i * bh, (i + 1) * bh)
        a2, l2 = _attn_sample(
            kis[i], hs_cur[sl], hd_cur[sl], W_q, gumbel[sl],
            nidx32[sl], nlen2[sl], cur2[sl],
        )
        acts.append(a2[:, 0])
        lps.append(l2[:, 0])
    return jnp.concatenate(acts), jnp.concatenate(lps)


# speedup vs baseline: 1.1151x; 1.0833x over previous
import functools
import math

import jax
import jax.numpy as jnp
import numpy as np
from jax import lax
from jax.experimental import pallas as pl
from jax.experimental.pallas import tpu as pltpu
from jax.experimental.pallas import tpu_sc as plsc

_GUMBEL_CACHE = {}


def _gumbel_const(b, m):
    if (b, m) not in _GUMBEL_CACHE:
        try:
            with jax.ensure_compile_time_eval():
                u = jax.random.uniform(jax.random.key(42), (b, m),
                                       minval=1e-9, maxval=1.0)
                g = -jnp.log(-jnp.log(u))
            _GUMBEL_CACHE[(b, m)] = np.asarray(g)
        except Exception:
            u = jax.random.uniform(jax.random.key(42), (b, m),
                                   minval=1e-9, maxval=1.0)
            return -jnp.log(-jnp.log(u))
    return jnp.asarray(_GUMBEL_CACHE[(b, m)])


def _bf16_bits(x_f32):
    b = jax.lax.bitcast_convert_type(x_f32.astype(jnp.bfloat16), jnp.uint16)
    return b.astype(jnp.int32)


def _kproj_body(hs_ref, hd_ref, wkse_ref, wkso_ref, wkde_ref, wkdo_ref,
                out_ref):
    hs = hs_ref[...].astype(jnp.bfloat16)
    hd = hd_ref[...].astype(jnp.bfloat16)
    ke = (jnp.dot(hs, wkse_ref[...].astype(jnp.bfloat16),
                  preferred_element_type=jnp.float32)
          + jnp.dot(hd, wkde_ref[...].astype(jnp.bfloat16),
                    preferred_element_type=jnp.float32))
    ko = (jnp.dot(hs, wkso_ref[...].astype(jnp.bfloat16),
                  preferred_element_type=jnp.float32)
          + jnp.dot(hd, wkdo_ref[...].astype(jnp.bfloat16),
                    preferred_element_type=jnp.float32))
    lo = _bf16_bits(ke)
    hi = _bf16_bits(ko)
    out_ref[...] = jax.lax.bitwise_or(jax.lax.shift_left(hi, 16), lo)


def _kproj(h_s, h_d, w_skvl, w_dkvl, block_n=1024):
    n, h = h_s.shape
    h2 = h // 2
    grid = (n // block_n,)
    return pl.pallas_call(
        _kproj_body,
        grid=grid,
        in_specs=[
            pl.BlockSpec((block_n, h), lambda i: (i, 0)),
            pl.BlockSpec((block_n, h), lambda i: (i, 0)),
            pl.BlockSpec((h, h2), lambda i: (0, 0)),
            pl.BlockSpec((h, h2), lambda i: (0, 1)),
            pl.BlockSpec((h, h2), lambda i: (0, 0)),
            pl.BlockSpec((h, h2), lambda i: (0, 1)),
        ],
        out_specs=pl.BlockSpec((block_n, h2), lambda i: (i, 0)),
        out_shape=jax.ShapeDtypeStruct((n, h2), jnp.int32),
    )(h_s, h_d, w_skvl, w_skvl, w_dkvl, w_dkvl)


def _sc_gather(table, idx_flat, chunk=128):
    n_rows = idx_flat.shape[0]
    h = table.shape[1]
    dt = table.dtype
    info = plsc.get_sparse_core_info()
    nw = info.num_cores * info.num_subcores
    rows_per_w = n_rows // nw
    n_chunks = rows_per_w // chunk
    mesh = plsc.VectorSubcoreMesh(core_axis_name="c", subcore_axis_name="s")

    nbuf = 3

    @functools.partial(
        pl.kernel,
        mesh=mesh,
        out_type=jax.ShapeDtypeStruct((n_rows, h), dt),
        scratch_types=[
            pltpu.VMEM((rows_per_w,), jnp.int32),
            pltpu.VMEM((chunk, h), dt),
            pltpu.VMEM((chunk, h), dt),
            pltpu.VMEM((chunk, h), dt),
            pltpu.SemaphoreType.DMA,
            pltpu.SemaphoreType.DMA,
            pltpu.SemaphoreType.DMA,
            pltpu.SemaphoreType.DMA,
        ],
    )
    def gather_kernel(table_hbm, idx_hbm, out_hbm, idx_v, rows_a, rows_b,
                      rows_c, gsem, osem_a, osem_b, osem_c):
        wid = lax.axis_index("s") * info.num_cores + lax.axis_index("c")
        base = wid * rows_per_w
        pltpu.sync_copy(idx_hbm.at[pl.ds(base, rows_per_w)], idx_v)
        bufs = (rows_a, rows_b, rows_c)
        osems = (osem_a, osem_b, osem_c)
        out_handles = [None] * nbuf
        for i in range(n_chunks):
            s = i % nbuf
            if out_handles[s] is not None:
                out_handles[s].wait()
            off = base + i * chunk
            pltpu.async_copy(
                table_hbm.at[idx_v.at[pl.ds(i * chunk, chunk)]], bufs[s], gsem
            ).wait()
            out_handles[s] = pltpu.async_copy(
                bufs[s], out_hbm.at[pl.ds(off, chunk)], osems[s])
        for hnd in out_handles:
            if hnd is not None:
                hnd.wait()

    return gather_kernel(table, idx_flat)


def _attn_body(ki_ref, hs_ref, hd_ref, wqe_ref, wqo_ref, g_ref, nidx_ref,
               nlen_ref, cur_ref, act_ref, lp_ref, *, m, h):
    bb = hs_ref.shape[0]
    hq = (hs_ref[...] + hd_ref[...]).astype(jnp.bfloat16)
    qe = jnp.dot(hq, wqe_ref[...].astype(jnp.bfloat16),
                 preferred_element_type=jnp.float32)
    qo = jnp.dot(hq, wqo_ref[...].astype(jnp.bfloat16),
                 preferred_element_type=jnp.float32)
    w = ki_ref[...]
    lo = jax.lax.bitcast_convert_type(
        jax.lax.shift_left(w, 16), jnp.float32)
    hi = jax.lax.bitcast_convert_type(
        jax.lax.bitwise_and(w, jnp.int32(-65536)), jnp.float32)
    qe_r = qe.astype(jnp.bfloat16).astype(jnp.float32)
    qo_r = qo.astype(jnp.bfloat16).astype(jnp.float32)
    compat = (
        jnp.sum(lo * qe_r[:, None, :], axis=-1)
        + jnp.sum(hi * qo_r[:, None, :], axis=-1)
    ) / math.sqrt(h)
    nlen = nlen_ref[...]
    lane = lax.broadcasted_iota(jnp.int32, (bb, m), 1)
    mask = lane < nlen
    logits = jnp.where(mask, compat, -1e9)
    z = logits + g_ref[...]
    idx = jnp.argmax(z, axis=1)
    mx = jnp.max(logits, axis=1, keepdims=True)
    shifted = logits - mx
    logp_all = shifted - jnp.log(jnp.sum(jnp.exp(shifted), axis=1, keepdims=True))
    sel = lane == idx[:, None]
    logp = jnp.sum(jnp.where(sel, logp_all, 0.0), axis=1)
    chosen = jnp.sum(jnp.where(sel, nidx_ref[...], 0), axis=1)
    empty = nlen[:, 0] == 0
    act_ref[...] = jnp.where(empty, cur_ref[...][:, 0], chosen)[:, None]
    lp_ref[...] = jnp.where(empty, 0.0, logp)[:, None]


def _attn_sample(ki, hs_cur, hd_cur, wq, gumbel, neigh_idx, neigh_len,
                 current_nodes, block_b=256):
    b, m = neigh_idx.shape
    h = hs_cur.shape[1]
    h2 = h // 2
    grid = (b // block_b,)
    return pl.pallas_call(
        functools.partial(_attn_body, m=m, h=h),
        grid=grid,
        in_specs=[
            pl.BlockSpec((block_b, m, h2), lambda i: (i, 0, 0)),
            pl.BlockSpec((block_b, h), lambda i: (i, 0)),
            pl.BlockSpec((block_b, h), lambda i: (i, 0)),
            pl.BlockSpec((h, h2), lambda i: (0, 0)),
            pl.BlockSpec((h, h2), lambda i: (0, 1)),
            pl.BlockSpec((block_b, m), lambda i: (i, 0)),
            pl.BlockSpec((block_b, m), lambda i: (i, 0)),
            pl.BlockSpec((block_b, 1), lambda i: (i, 0)),
            pl.BlockSpec((block_b, 1), lambda i: (i, 0)),
        ],
        out_specs=[
            pl.BlockSpec((block_b, 1), lambda i: (i, 0)),
            pl.BlockSpec((block_b, 1), lambda i: (i, 0)),
        ],
        out_shape=[
            jax.ShapeDtypeStruct((b, 1), jnp.int32),
            jax.ShapeDtypeStruct((b, 1), jnp.float32),
        ],
    )(ki, hs_cur, hd_cur, wq, wq, gumbel, neigh_idx, neigh_len,
      current_nodes)


def kernel(h_dynamic, h_static, W_skvl, W_dkvl, W_q, current_nodes, neigh_idx,
           neigh_len):
    n, h = h_static.shape
    b, m = neigh_idx.shape
    kall_i32 = _kproj(h_static, h_dynamic, W_skvl, W_dkvl)

    cur = current_nodes.astype(jnp.int32)
    hs_cur = h_static[cur]
    hd_cur = h_dynamic[cur]

    gumbel = _gumbel_const(b, m)

    idx_flat = neigh_idx.reshape(b * m).astype(jnp.int32)
    nidx32 = neigh_idx.astype(jnp.int32)
    nlen2 = neigh_len.reshape(b, 1).astype(jnp.int32)
    cur2 = cur.reshape(b, 1)

    nsplit = 2
    bh = b // nsplit
    acts, lps = [], []
    kis = [
        _sc_gather(kall_i32, idx_flat[i * bh * m:(i + 1) * bh * m])
        .reshape(bh, m, h // 2)
        for i in range(nsplit)
    ]
    for i in range(nsplit):
        sl = slice(i * bh, (i + 1) * bh)
        a2, l2 = _attn_sample(
            kis[i], hs_cur[sl], hd_cur[sl], W_q, gumbel[sl],
            nidx32[sl], nlen2[sl], cur2[sl],
        )
        acts.append(a2[:, 0])
        lps.append(l2[:, 0])
    return jnp.concatenate(acts), jnp.concatenate(lps)
